# Initial kernel scaffold; baseline (speedup 1.0000x reference)
import jax
import jax.numpy as jnp
from jax.experimental import pallas as pl


def kernel(inputs, label, iou, lut, cq):
    raise NotImplementedError("write your pallas kernel here")


# speedup vs baseline: 9.3463x; 9.3463x over previous
import jax
import jax.numpy as jnp
from jax.experimental import pallas as pl
from jax.experimental.pallas import tpu as pltpu

_F = 128
_NPIDS = 100000
_NCQ = 5000
_SCALE = 30.0
_B = 1024

_NL_TILE = 2000
_KL = _NPIDS // _NL_TILE
_NC_TILE = 1000
_KC = _NCQ // _NC_TILE
_K = _KL + _KC


def _body(x_ref, lab_ref, iou_ref, lut_ref, cq_ref, out_ref,
          xh_ref, m_ref, s_ref, t_ref):
    k = pl.program_id(0)

    @pl.when(k == 0)
    def _init():
        x = x_ref[...]
        ss = jnp.sum(x * x, axis=1, keepdims=True)
        nrm = jnp.maximum(jnp.sqrt(ss), 1e-12)
        xh_ref[...] = x * (_SCALE / nrm)
        m_ref[...] = jnp.full((_B, 1), -1e30, jnp.float32)
        s_ref[...] = jnp.zeros((_B, 1), jnp.float32)
        t_ref[...] = jnp.zeros((_B, 1), jnp.float32)

    def _process(tbl, n_tile, base, with_fix):
        xh = xh_ref[...]
        ones = jnp.ones((1, _F), jnp.float32)
        z = jax.lax.dot_general(ones, jnp.abs(tbl),
                                (((1,), (1,)), ((), ())),
                                preferred_element_type=jnp.float32)
        bad = z == 0.0
        p = jax.lax.dot_general(xh, tbl, (((1,), (1,)), ((), ())),
                                preferred_element_type=jnp.float32)
        p = jnp.where(bad, -_SCALE, p)
        if with_fix:
            cols = base + jax.lax.broadcasted_iota(jnp.int32, (_B, n_tile), 1)
            match = cols == lab_ref[...]
            p = jnp.where(jnp.logical_and(match, bad), _SCALE, p)
            t_ref[...] += jnp.sum(jnp.where(match, p, 0.0), axis=1,
                                  keepdims=True)
        tmax = jnp.max(p, axis=1, keepdims=True)
        m_old = m_ref[...]
        m_new = jnp.maximum(m_old, tmax)
        s_ref[...] = (s_ref[...] * jnp.exp(m_old - m_new)
                      + jnp.sum(jnp.exp(p - m_new), axis=1, keepdims=True))
        m_ref[...] = m_new

    @pl.when(k < _KL)
    def _lut_step():
        _process(lut_ref[...], _NL_TILE, k * _NL_TILE, True)

    @pl.when(k >= _KL)
    def _cq_step():
        _process(cq_ref[...], _NC_TILE, 0, False)

    @pl.when(k == _K - 1)
    def _finish():
        lab = lab_ref[...]
        iou = iou_ref[...]
        lse = m_ref[...] + jnp.log(s_ref[...])
        valid = lab < _NPIDS
        terms = jnp.where(valid, iou * iou * (lse - t_ref[...]), 0.0)
        nvalid = jnp.sum(jnp.where(valid, 1.0, 0.0), axis=(0, 1),
                         keepdims=True)
        out_ref[...] = jnp.sum(terms, axis=(0, 1), keepdims=True) / nvalid


def kernel(inputs, label, iou, lut, cq):
    lab2 = label.reshape(_B, 1)
    iou2 = iou.reshape(_B, 1)
    out = pl.pallas_call(
        _body,
        grid=(_K,),
        in_specs=[
            pl.BlockSpec((_B, _F), lambda k: (0, 0)),
            pl.BlockSpec((_B, 1), lambda k: (0, 0)),
            pl.BlockSpec((_B, 1), lambda k: (0, 0)),
            pl.BlockSpec((_NL_TILE, _F),
                         lambda k: (jnp.minimum(k, _KL - 1), 0)),
            pl.BlockSpec((_NC_TILE, _F),
                         lambda k: (jnp.maximum(k - _KL, 0), 0)),
        ],
        out_specs=pl.BlockSpec((1, 1), lambda k: (0, 0)),
        out_shape=jax.ShapeDtypeStruct((1, 1), jnp.float32),
        scratch_shapes=[
            pltpu.VMEM((_B, _F), jnp.float32),
            pltpu.VMEM((_B, 1), jnp.float32),
            pltpu.VMEM((_B, 1), jnp.float32),
            pltpu.VMEM((_B, 1), jnp.float32),
        ],
        compiler_params=pltpu.CompilerParams(
            dimension_semantics=("arbitrary",),
        ),
    )(inputs, lab2, iou2, lut, cq)
    return out[0, 0]


# speedup vs baseline: 11.1319x; 1.1910x over previous
import jax
import jax.numpy as jnp
from jax import lax
from jax.experimental import pallas as pl
from jax.experimental.pallas import tpu as pltpu
from jax.experimental.pallas import tpu_sc as plsc

_F = 128
_NPIDS = 100000
_NCQ = 5000
_SCALE = 30.0
_B = 1024

_NL_TILE = 2000
_KL = _NPIDS // _NL_TILE
_NC_TILE = 1000
_KC = _NCQ // _NC_TILE
_K = _KL + _KC

_NW = 32
_BPW = _B // _NW


def _sc_gather_body(table_hbm, idx_hbm, out_hbm, idx_v, rows_v, sem):
    wid = lax.axis_index("s") * 2 + lax.axis_index("c")
    base = wid * _BPW
    pltpu.sync_copy(idx_hbm.at[pl.ds(base, _BPW)], idx_v)
    pltpu.async_copy(table_hbm.at[idx_v], rows_v, sem).wait()
    pltpu.sync_copy(rows_v, out_hbm.at[pl.ds(base, _BPW)])


def _gather_label_rows(lut, idx):
    mesh = plsc.VectorSubcoreMesh(core_axis_name="c", subcore_axis_name="s")
    return pl.kernel(
        _sc_gather_body,
        mesh=mesh,
        out_type=jax.ShapeDtypeStruct((_B, _F), jnp.float32),
        scratch_types=[
            pltpu.VMEM((_BPW,), jnp.int32),
            pltpu.VMEM((_BPW, _F), jnp.float32),
            pltpu.SemaphoreType.DMA,
        ],
    )(lut, idx)


def _body(x_ref, lab_ref, iou_ref, g_ref, lut_ref, cq_ref, out_ref,
          xh_ref, m_ref, s_ref):
    k = pl.program_id(0)

    @pl.when(k == 0)
    def _init():
        x = x_ref[...]
        ss = jnp.sum(x * x, axis=1, keepdims=True)
        nrm = jnp.maximum(jnp.sqrt(ss), 1e-12)
        xh_ref[...] = x * (_SCALE / nrm)
        m_ref[...] = jnp.full((_B, 1), -1e30, jnp.float32)
        s_ref[...] = jnp.zeros((_B, 1), jnp.float32)

    def _process(tbl):
        xh = xh_ref[...]
        ones = jnp.ones((1, _F), jnp.float32)
        z = jax.lax.dot_general(ones, jnp.abs(tbl),
                                (((1,), (1,)), ((), ())),
                                preferred_element_type=jnp.float32)
        bad = z == 0.0
        p = jax.lax.dot_general(xh, tbl, (((1,), (1,)), ((), ())),
                                preferred_element_type=jnp.float32)
        p = jnp.where(bad, -_SCALE, p)
        tmax = jnp.max(p, axis=1, keepdims=True)
        m_old = m_ref[...]
        m_new = jnp.maximum(m_old, tmax)
        s_ref[...] = (s_ref[...] * jnp.exp(m_old - m_new)
                      + jnp.sum(jnp.exp(p - m_new), axis=1, keepdims=True))
        m_ref[...] = m_new

    @pl.when(k < _KL)
    def _lut_step():
        _process(lut_ref[...])

    @pl.when(k >= _KL)
    def _cq_step():
        _process(cq_ref[...])

    @pl.when(k == _K - 1)
    def _finish():
        lab = lab_ref[...]
        iou = iou_ref[...]
        g = g_ref[...]
        xh = xh_ref[...]
        d = jnp.sum(xh * g, axis=1, keepdims=True)
        gz = jnp.sum(jnp.abs(g), axis=1, keepdims=True)
        bad_l = gz == 0.0
        a = jnp.where(bad_l, -_SCALE, d)
        b = jnp.where(bad_l, _SCALE, d)
        m_raw = m_ref[...]
        s_raw = s_ref[...]
        m2 = jnp.maximum(m_raw, b)
        s2 = (s_raw * jnp.exp(m_raw - m2)
              - jnp.exp(a - m2) + jnp.exp(b - m2))
        lse = m2 + jnp.log(s2)
        valid = lab < _NPIDS
        terms = jnp.where(valid, iou * iou * (lse - b), 0.0)
        nvalid = jnp.sum(jnp.where(valid, 1.0, 0.0), axis=(0, 1),
                         keepdims=True)
        out_ref[...] = jnp.sum(terms, axis=(0, 1), keepdims=True) / nvalid


def kernel(inputs, label, iou, lut, cq):
    lab_safe = jnp.clip(label, 0, _NPIDS - 1)
    g = _gather_label_rows(lut, lab_safe)
    lab2 = label.reshape(_B, 1)
    iou2 = iou.reshape(_B, 1)
    out = pl.pallas_call(
        _body,
        grid=(_K,),
        in_specs=[
            pl.BlockSpec((_B, _F), lambda k: (0, 0)),
            pl.BlockSpec((_B, 1), lambda k: (0, 0)),
            pl.BlockSpec((_B, 1), lambda k: (0, 0)),
            pl.BlockSpec((_B, _F), lambda k: (0, 0)),
            pl.BlockSpec((_NL_TILE, _F),
                         lambda k: (jnp.minimum(k, _KL - 1), 0)),
            pl.BlockSpec((_NC_TILE, _F),
                         lambda k: (jnp.maximum(k - _KL, 0), 0)),
        ],
        out_specs=pl.BlockSpec((1, 1), lambda k: (0, 0)),
        out_shape=jax.ShapeDtypeStruct((1, 1), jnp.float32),
        scratch_shapes=[
            pltpu.VMEM((_B, _F), jnp.float32),
            pltpu.VMEM((_B, 1), jnp.float32),
            pltpu.VMEM((_B, 1), jnp.float32),
        ],
        compiler_params=pltpu.CompilerParams(
            dimension_semantics=("arbitrary",),
        ),
    )(inputs, lab2, iou2, g, lut, cq)
    return out[0, 0]
